# baseline (speedup 1.0000x reference)
import functools

import jax
import jax.numpy as jnp
from jax import lax
from jax.experimental import pallas as pl
from jax.experimental.pallas import tpu as pltpu
from jax.experimental.pallas import tpu_sc as plsc

B, S, D = 1, 2048, 1024
E, TOPK, H = 8, 2, 8
DK = DV = D // H
CAP = S // TOPK
C = 512
N = CAP // C
EPS = 1e-5
PAD = 8
TRASH = E * CAP
NW = 32
TPW = S // NW
NEG = -1e30


def _elu1(x):
    return jnp.where(x > 0, x + 1.0, jnp.exp(jnp.minimum(x, 0.0)))


def _bf(x):
    return x.astype(jnp.bfloat16)


def _router_body(x_ref, wg_ref, dstA_ref, dstB_ref, cmbA_ref, cmbB_ref,
                 wA_ref, wB_ref, cval_ref):
    lg = lax.dot_general(wg_ref[...], x_ref[...],
                         (((1,), (1,)), ((), ())),
                         preferred_element_type=jnp.float32)
    ie = lax.broadcasted_iota(jnp.int32, (E, S), 0)
    m1 = jnp.max(lg, axis=0, keepdims=True)
    i1 = jnp.min(jnp.where(lg == m1, ie, E), axis=0, keepdims=True)
    lg2 = jnp.where(ie == i1, NEG, lg)
    m2 = jnp.max(lg2, axis=0, keepdims=True)
    i2 = jnp.min(jnp.where(lg2 == m2, ie, E), axis=0, keepdims=True)
    wA = 1.0 / (1.0 + jnp.exp(m2 - m1))
    wB = 1.0 - wA
    oh1 = (ie == i1)
    oh2 = (ie == i2)
    M = jnp.where(jnp.logical_or(oh1, oh2), 1, 0)
    inc = M
    sh = 1
    while sh < S:
        inc = inc + jnp.concatenate(
            [jnp.zeros((E, sh), jnp.int32), inc[:, : S - sh]], axis=1)
        sh *= 2
    exc = inc - M
    count = inc[:, S - 1 : S]
    over = jnp.maximum(count - CAP, 0)
    cval_ref[...] = jnp.minimum(count, CAP)
    oh1f = jnp.where(oh1, 1, 0)
    oh2f = jnp.where(oh2, 1, 0)
    r0 = jnp.sum(exc * oh1f, axis=0, keepdims=True)
    r1 = jnp.sum(exc * oh2f, axis=0, keepdims=True)
    ov0 = jnp.sum(over * oh1f, axis=0, keepdims=True)
    ov1 = jnp.sum(over * oh2f, axis=0, keepdims=True)
    s0 = r0 - ov0
    s1 = r1 - ov1
    d0 = i1 * CAP + s0
    d1 = i2 * CAP + s1
    val0 = s0 >= 0
    val1 = s1 >= 0
    dstA_ref[...] = jnp.where(val0, d0, TRASH)
    dstB_ref[...] = jnp.where(val1, d1, TRASH)
    cmbA_ref[...] = jnp.where(val0, d0, 0)
    cmbB_ref[...] = jnp.where(val1, d1, 0)
    wA_ref[...] = jnp.where(val0, wA, 0.0)
    wB_ref[...] = jnp.where(val1, wB, 0.0)


def _router(x, Wg):
    i32 = jnp.int32
    f32 = jnp.float32
    return pl.pallas_call(
        _router_body,
        out_shape=[
            jax.ShapeDtypeStruct((1, S), i32),
            jax.ShapeDtypeStruct((1, S), i32),
            jax.ShapeDtypeStruct((1, S), i32),
            jax.ShapeDtypeStruct((1, S), i32),
            jax.ShapeDtypeStruct((1, S), f32),
            jax.ShapeDtypeStruct((1, S), f32),
            jax.ShapeDtypeStruct((E, 1), i32),
        ],
    )(x, Wg)


def _qproj_body(x_ref, wq_ref, q_ref):
    q_ref[...] = _elu1(
        lax.dot_general(_bf(x_ref[...]), _bf(wq_ref[...]),
                        (((1,), (1,)), ((), ())),
                        preferred_element_type=jnp.float32))


def _qproj(x, Wq):
    TB = 256
    return pl.pallas_call(
        _qproj_body,
        grid=(S // TB,),
        in_specs=[
            pl.BlockSpec((TB, D), lambda i: (i, 0)),
            pl.BlockSpec((D, D), lambda i: (0, 0)),
        ],
        out_specs=pl.BlockSpec((TB, D), lambda i: (i, 0)),
        out_shape=jax.ShapeDtypeStruct((S, D), jnp.float32),
    )(x, Wq)


def _dispatch_body(x_hbm, q_hbm, dstA_hbm, dstB_hbm, gx_hbm, gq_hbm,
                   idxA, idxB, rows, sem):
    wid = lax.axis_index("s") * 2 + lax.axis_index("c")
    base = wid * TPW
    pltpu.sync_copy(dstA_hbm.at[pl.ds(base, TPW)], idxA)
    pltpu.sync_copy(dstB_hbm.at[pl.ds(base, TPW)], idxB)
    pltpu.sync_copy(x_hbm.at[pl.ds(base, TPW)], rows)
    pltpu.async_copy(rows, gx_hbm.at[idxA], sem).wait()
    pltpu.async_copy(rows, gx_hbm.at[idxB], sem).wait()
    pltpu.sync_copy(q_hbm.at[pl.ds(base, TPW)], rows)
    pltpu.async_copy(rows, gq_hbm.at[idxA], sem).wait()
    pltpu.async_copy(rows, gq_hbm.at[idxB], sem).wait()


def _dispatch(x, q, dstA, dstB):
    mesh = plsc.VectorSubcoreMesh(core_axis_name="c", subcore_axis_name="s")
    i32 = jnp.int32
    f32 = jnp.float32
    fn = functools.partial(
        pl.kernel,
        out_type=[
            jax.ShapeDtypeStruct((E * CAP + PAD, D), f32),
            jax.ShapeDtypeStruct((E * CAP + PAD, D), f32),
        ],
        mesh=mesh,
        scratch_types=[
            pltpu.VMEM((TPW,), i32),
            pltpu.VMEM((TPW,), i32),
            pltpu.VMEM((TPW, D), f32),
            pltpu.SemaphoreType.DMA,
        ],
    )(_dispatch_body)
    return fn(x, q, dstA, dstB)


def _attn_body(cval_ref, gx_ref, gq_ref, wk_ref, wv_ref, o_ref, st_ref,
               wkb_ref, wvb_ref):
    n = pl.program_id(1)
    e = pl.program_id(0)
    v = cval_ref[e, 0]

    @pl.when(jnp.logical_or(n == 0, n * C < v))
    def _():
        @pl.when(n == 0)
        def _():
            st_ref[...] = jnp.zeros((H, DK, DV), jnp.float32)
            wkb_ref[...] = _bf(wk_ref[0])
            wvb_ref[...] = _bf(wv_ref[0])

        xb = _bf(gx_ref[...])
        k = _elu1(lax.dot_general(xb, wkb_ref[...], (((1,), (1,)), ((), ())),
                                  preferred_element_type=jnp.float32))
        vv = lax.dot_general(xb, wvb_ref[...], (((1,), (1,)), ((), ())),
                             preferred_element_type=jnp.float32)
        rv = (n * C + lax.broadcasted_iota(jnp.int32, (C, 1), 0)) < v
        k = _bf(jnp.where(rv, k, 0.0))
        vb = _bf(jnp.where(rv, vv, 0.0))
        qb = _bf(gq_ref[...])
        ri = lax.broadcasted_iota(jnp.int32, (C, C), 0)
        ci = lax.broadcasted_iota(jnp.int32, (C, C), 1)
        tri = ri >= ci
        outs = []
        for h in range(H):
            qh = qb[:, h * DK : (h + 1) * DK]
            kh = k[:, h * DK : (h + 1) * DK]
            vh = vb[:, h * DV : (h + 1) * DV]
            sth = st_ref[h]
            inter = lax.dot_general(qh, _bf(sth), (((1,), (0,)), ((), ())),
                                    preferred_element_type=jnp.float32)
            attn = lax.dot_general(qh, kh, (((1,), (1,)), ((), ())),
                                   preferred_element_type=jnp.float32)
            attn = _bf(jnp.where(tri, attn, 0.0))
            intra = lax.dot_general(attn, vh, (((1,), (0,)), ((), ())),
                                    preferred_element_type=jnp.float32)
            outs.append(inter + intra)
            st_ref[h] = sth + lax.dot_general(
                kh, vh, (((0,), (0,)), ((), ())),
                preferred_element_type=jnp.float32)
        o_ref[...] = jnp.concatenate(outs, axis=1)


def _attention(cvalid, gx, gq, Wk, Wv):
    return pl.pallas_call(
        _attn_body,
        grid=(E, N),
        in_specs=[
            pl.BlockSpec(memory_space=pltpu.SMEM),
            pl.BlockSpec((C, D), lambda e, n: (e * N + n, 0)),
            pl.BlockSpec((C, D), lambda e, n: (e * N + n, 0)),
            pl.BlockSpec((1, D, D), lambda e, n: (e, 0, 0)),
            pl.BlockSpec((1, D, D), lambda e, n: (e, 0, 0)),
        ],
        out_specs=pl.BlockSpec((C, D), lambda e, n: (e * N + n, 0)),
        out_shape=jax.ShapeDtypeStruct((E * CAP, D), jnp.float32),
        scratch_shapes=[pltpu.VMEM((H, DK, DV), jnp.float32),
                        pltpu.VMEM((D, D), jnp.bfloat16),
                        pltpu.VMEM((D, D), jnp.bfloat16)],
    )(cvalid, gx, gq, Wk, Wv)


def _combine_body(o_hbm, cmbA_hbm, cmbB_hbm, rA_hbm, rB_hbm, idx, rows, sem):
    wid = lax.axis_index("s") * 2 + lax.axis_index("c")
    base = wid * TPW
    pltpu.sync_copy(cmbA_hbm.at[pl.ds(base, TPW)], idx)
    pltpu.async_copy(o_hbm.at[idx], rows, sem).wait()
    pltpu.sync_copy(rows, rA_hbm.at[pl.ds(base, TPW)])
    pltpu.sync_copy(cmbB_hbm.at[pl.ds(base, TPW)], idx)
    pltpu.async_copy(o_hbm.at[idx], rows, sem).wait()
    pltpu.sync_copy(rows, rB_hbm.at[pl.ds(base, TPW)])


def _combine(o_exp, cmbA, cmbB):
    mesh = plsc.VectorSubcoreMesh(core_axis_name="c", subcore_axis_name="s")
    i32 = jnp.int32
    f32 = jnp.float32
    fn = functools.partial(
        pl.kernel,
        out_type=[
            jax.ShapeDtypeStruct((S, D), f32),
            jax.ShapeDtypeStruct((S, D), f32),
        ],
        mesh=mesh,
        scratch_types=[
            pltpu.VMEM((TPW,), i32),
            pltpu.VMEM((TPW, D), f32),
            pltpu.SemaphoreType.DMA,
        ],
    )(_combine_body)
    return fn(o_exp, cmbA, cmbB)


def _final_body(ra_ref, rb_ref, wa_ref, wb_ref, g_ref, wo_ref, out_ref):
    y = ra_ref[...] * wa_ref[...] + rb_ref[...] * wb_ref[...]
    ms = jnp.mean(y * y, axis=1, keepdims=True)
    y = y * lax.rsqrt(ms + EPS) * g_ref[...]
    out_ref[...] = lax.dot_general(_bf(y), _bf(wo_ref[...]),
                                   (((1,), (1,)), ((), ())),
                                   preferred_element_type=jnp.float32)


def _final(rowsA, rowsB, wA, wB, g_norm, Wo):
    TB = 256
    return pl.pallas_call(
        _final_body,
        grid=(S // TB,),
        in_specs=[
            pl.BlockSpec((TB, D), lambda i: (i, 0)),
            pl.BlockSpec((TB, D), lambda i: (i, 0)),
            pl.BlockSpec((TB, 1), lambda i: (i, 0)),
            pl.BlockSpec((TB, 1), lambda i: (i, 0)),
            pl.BlockSpec((1, D), lambda i: (0, 0)),
            pl.BlockSpec((D, D), lambda i: (0, 0)),
        ],
        out_specs=pl.BlockSpec((TB, D), lambda i: (i, 0)),
        out_shape=jax.ShapeDtypeStruct((S, D), jnp.float32),
    )(rowsA, rowsB, wA, wB, g_norm, Wo)


def kernel(hidden_states, Wq, Wk, Wv, Wg, Wo, g_norm):
    x = hidden_states[0]
    dstA, dstB, cmbA, cmbB, wA, wB, cvalid = _router(x, Wg)
    q = _qproj(x, Wq)
    gx, gq = _dispatch(x, q, dstA.reshape(S), dstB.reshape(S))
    o_exp = _attention(cvalid, gx, gq, Wk, Wv)
    rowsA, rowsB = _combine(o_exp, cmbA.reshape(S), cmbB.reshape(S))
    out = _final(rowsA, rowsB, wA.reshape(S, 1), wB.reshape(S, 1),
                 g_norm.reshape(1, D), Wo)
    return out[None]


# speedup vs baseline: 1.0190x; 1.0190x over previous
import functools

import jax
import jax.numpy as jnp
from jax import lax
from jax.experimental import pallas as pl
from jax.experimental.pallas import tpu as pltpu
from jax.experimental.pallas import tpu_sc as plsc

B, S, D = 1, 2048, 1024
E, TOPK, H = 8, 2, 8
DK = DV = D // H
CAP = S // TOPK
C = 256
N = CAP // C
EPS = 1e-5
PAD = 8
TRASH = E * CAP
NW = 32
TPW = S // NW
NEG = -1e30


def _elu1(x):
    return jnp.where(x > 0, x + 1.0, jnp.exp(jnp.minimum(x, 0.0)))


def _bf(x):
    return x.astype(jnp.bfloat16)


def _router_body(x_ref, wg_ref, dstA_ref, dstB_ref, cmbA_ref, cmbB_ref,
                 wA_ref, wB_ref, cval_ref):
    lg = lax.dot_general(wg_ref[...], x_ref[...],
                         (((1,), (1,)), ((), ())),
                         preferred_element_type=jnp.float32)
    ie = lax.broadcasted_iota(jnp.int32, (E, S), 0)
    m1 = jnp.max(lg, axis=0, keepdims=True)
    i1 = jnp.min(jnp.where(lg == m1, ie, E), axis=0, keepdims=True)
    lg2 = jnp.where(ie == i1, NEG, lg)
    m2 = jnp.max(lg2, axis=0, keepdims=True)
    i2 = jnp.min(jnp.where(lg2 == m2, ie, E), axis=0, keepdims=True)
    wA = 1.0 / (1.0 + jnp.exp(m2 - m1))
    wB = 1.0 - wA
    oh1 = (ie == i1)
    oh2 = (ie == i2)
    M = jnp.where(jnp.logical_or(oh1, oh2), 1, 0)
    inc = M
    sh = 1
    while sh < S:
        inc = inc + jnp.concatenate(
            [jnp.zeros((E, sh), jnp.int32), inc[:, : S - sh]], axis=1)
        sh *= 2
    exc = inc - M
    count = inc[:, S - 1 : S]
    over = jnp.maximum(count - CAP, 0)
    cval_ref[...] = jnp.minimum(count, CAP)
    oh1f = jnp.where(oh1, 1, 0)
    oh2f = jnp.where(oh2, 1, 0)
    r0 = jnp.sum(exc * oh1f, axis=0, keepdims=True)
    r1 = jnp.sum(exc * oh2f, axis=0, keepdims=True)
    ov0 = jnp.sum(over * oh1f, axis=0, keepdims=True)
    ov1 = jnp.sum(over * oh2f, axis=0, keepdims=True)
    s0 = r0 - ov0
    s1 = r1 - ov1
    d0 = i1 * CAP + s0
    d1 = i2 * CAP + s1
    val0 = s0 >= 0
    val1 = s1 >= 0
    dstA_ref[...] = jnp.where(val0, d0, TRASH)
    dstB_ref[...] = jnp.where(val1, d1, TRASH)
    cmbA_ref[...] = jnp.where(val0, d0, 0)
    cmbB_ref[...] = jnp.where(val1, d1, 0)
    wA_ref[...] = jnp.where(val0, wA, 0.0)
    wB_ref[...] = jnp.where(val1, wB, 0.0)


def _router(x, Wg):
    i32 = jnp.int32
    f32 = jnp.float32
    return pl.pallas_call(
        _router_body,
        out_shape=[
            jax.ShapeDtypeStruct((1, S), i32),
            jax.ShapeDtypeStruct((1, S), i32),
            jax.ShapeDtypeStruct((1, S), i32),
            jax.ShapeDtypeStruct((1, S), i32),
            jax.ShapeDtypeStruct((1, S), f32),
            jax.ShapeDtypeStruct((1, S), f32),
            jax.ShapeDtypeStruct((E, 1), i32),
        ],
    )(x, Wg)


def _qproj_body(x_ref, wq_ref, q_ref):
    q_ref[...] = _elu1(
        lax.dot_general(_bf(x_ref[...]), _bf(wq_ref[...]),
                        (((1,), (1,)), ((), ())),
                        preferred_element_type=jnp.float32))


def _qproj(x, Wq):
    TB = 256
    return pl.pallas_call(
        _qproj_body,
        grid=(S // TB,),
        in_specs=[
            pl.BlockSpec((TB, D), lambda i: (i, 0)),
            pl.BlockSpec((D, D), lambda i: (0, 0)),
        ],
        out_specs=pl.BlockSpec((TB, D), lambda i: (i, 0)),
        out_shape=jax.ShapeDtypeStruct((S, D), jnp.float32),
    )(x, Wq)


def _dispatch_body(x_hbm, q_hbm, dstA_hbm, dstB_hbm, gx_hbm, gq_hbm,
                   idxA, idxB, rows, sem):
    wid = lax.axis_index("s") * 2 + lax.axis_index("c")
    base = wid * TPW
    pltpu.sync_copy(dstA_hbm.at[pl.ds(base, TPW)], idxA)
    pltpu.sync_copy(dstB_hbm.at[pl.ds(base, TPW)], idxB)
    pltpu.sync_copy(x_hbm.at[pl.ds(base, TPW)], rows)
    pltpu.async_copy(rows, gx_hbm.at[idxA], sem).wait()
    pltpu.async_copy(rows, gx_hbm.at[idxB], sem).wait()
    pltpu.sync_copy(q_hbm.at[pl.ds(base, TPW)], rows)
    pltpu.async_copy(rows, gq_hbm.at[idxA], sem).wait()
    pltpu.async_copy(rows, gq_hbm.at[idxB], sem).wait()


def _dispatch(x, q, dstA, dstB):
    mesh = plsc.VectorSubcoreMesh(core_axis_name="c", subcore_axis_name="s")
    i32 = jnp.int32
    f32 = jnp.float32
    fn = functools.partial(
        pl.kernel,
        out_type=[
            jax.ShapeDtypeStruct((E * CAP + PAD, D), f32),
            jax.ShapeDtypeStruct((E * CAP + PAD, D), f32),
        ],
        mesh=mesh,
        scratch_types=[
            pltpu.VMEM((TPW,), i32),
            pltpu.VMEM((TPW,), i32),
            pltpu.VMEM((TPW, D), f32),
            pltpu.SemaphoreType.DMA,
        ],
    )(_dispatch_body)
    return fn(x, q, dstA, dstB)


def _attn_body(cval_ref, gx_ref, gq_ref, wk_ref, wv_ref, o_ref, st_ref,
               wkb_ref, wvb_ref):
    n = pl.program_id(1)
    e = pl.program_id(0)
    v = cval_ref[e, 0]

    @pl.when(jnp.logical_or(n == 0, n * C < v))
    def _():
        @pl.when(n == 0)
        def _():
            st_ref[...] = jnp.zeros((H, DK, DV), jnp.float32)
            wkb_ref[...] = _bf(wk_ref[0])
            wvb_ref[...] = _bf(wv_ref[0])

        xb = _bf(gx_ref[...])
        k = _elu1(lax.dot_general(xb, wkb_ref[...], (((1,), (1,)), ((), ())),
                                  preferred_element_type=jnp.float32))
        vv = lax.dot_general(xb, wvb_ref[...], (((1,), (1,)), ((), ())),
                             preferred_element_type=jnp.float32)
        rv = (n * C + lax.broadcasted_iota(jnp.int32, (C, 1), 0)) < v
        k = _bf(jnp.where(rv, k, 0.0))
        vb = _bf(jnp.where(rv, vv, 0.0))
        qb = _bf(gq_ref[...])
        ri = lax.broadcasted_iota(jnp.int32, (C, C), 0)
        ci = lax.broadcasted_iota(jnp.int32, (C, C), 1)
        tri = ri >= ci
        outs = []
        for h in range(H):
            qh = qb[:, h * DK : (h + 1) * DK]
            kh = k[:, h * DK : (h + 1) * DK]
            vh = vb[:, h * DV : (h + 1) * DV]
            sth = st_ref[h]
            inter = lax.dot_general(qh, _bf(sth), (((1,), (0,)), ((), ())),
                                    preferred_element_type=jnp.float32)
            attn = lax.dot_general(qh, kh, (((1,), (1,)), ((), ())),
                                   preferred_element_type=jnp.float32)
            attn = _bf(jnp.where(tri, attn, 0.0))
            intra = lax.dot_general(attn, vh, (((1,), (0,)), ((), ())),
                                    preferred_element_type=jnp.float32)
            outs.append(inter + intra)
            st_ref[h] = sth + lax.dot_general(
                kh, vh, (((0,), (0,)), ((), ())),
                preferred_element_type=jnp.float32)
        o_ref[...] = jnp.concatenate(outs, axis=1)


def _attention(cvalid, gx, gq, Wk, Wv):
    return pl.pallas_call(
        _attn_body,
        grid=(E, N),
        in_specs=[
            pl.BlockSpec(memory_space=pltpu.SMEM),
            pl.BlockSpec((C, D), lambda e, n: (e * N + n, 0)),
            pl.BlockSpec((C, D), lambda e, n: (e * N + n, 0)),
            pl.BlockSpec((1, D, D), lambda e, n: (e, 0, 0)),
            pl.BlockSpec((1, D, D), lambda e, n: (e, 0, 0)),
        ],
        out_specs=pl.BlockSpec((C, D), lambda e, n: (e * N + n, 0)),
        out_shape=jax.ShapeDtypeStruct((E * CAP, D), jnp.float32),
        scratch_shapes=[pltpu.VMEM((H, DK, DV), jnp.float32),
                        pltpu.VMEM((D, D), jnp.bfloat16),
                        pltpu.VMEM((D, D), jnp.bfloat16)],
    )(cvalid, gx, gq, Wk, Wv)


def _combine_body(o_hbm, cmbA_hbm, cmbB_hbm, rA_hbm, rB_hbm, idx, rows, sem):
    wid = lax.axis_index("s") * 2 + lax.axis_index("c")
    base = wid * TPW
    pltpu.sync_copy(cmbA_hbm.at[pl.ds(base, TPW)], idx)
    pltpu.async_copy(o_hbm.at[idx], rows, sem).wait()
    pltpu.sync_copy(rows, rA_hbm.at[pl.ds(base, TPW)])
    pltpu.sync_copy(cmbB_hbm.at[pl.ds(base, TPW)], idx)
    pltpu.async_copy(o_hbm.at[idx], rows, sem).wait()
    pltpu.sync_copy(rows, rB_hbm.at[pl.ds(base, TPW)])


def _combine(o_exp, cmbA, cmbB):
    mesh = plsc.VectorSubcoreMesh(core_axis_name="c", subcore_axis_name="s")
    i32 = jnp.int32
    f32 = jnp.float32
    fn = functools.partial(
        pl.kernel,
        out_type=[
            jax.ShapeDtypeStruct((S, D), f32),
            jax.ShapeDtypeStruct((S, D), f32),
        ],
        mesh=mesh,
        scratch_types=[
            pltpu.VMEM((TPW,), i32),
            pltpu.VMEM((TPW, D), f32),
            pltpu.SemaphoreType.DMA,
        ],
    )(_combine_body)
    return fn(o_exp, cmbA, cmbB)


def _final_body(ra_ref, rb_ref, wa_ref, wb_ref, g_ref, wo_ref, out_ref):
    y = ra_ref[...] * wa_ref[...] + rb_ref[...] * wb_ref[...]
    ms = jnp.mean(y * y, axis=1, keepdims=True)
    y = y * lax.rsqrt(ms + EPS) * g_ref[...]
    out_ref[...] = lax.dot_general(_bf(y), _bf(wo_ref[...]),
                                   (((1,), (1,)), ((), ())),
                                   preferred_element_type=jnp.float32)


def _final(rowsA, rowsB, wA, wB, g_norm, Wo):
    TB = 256
    return pl.pallas_call(
        _final_body,
        grid=(S // TB,),
        in_specs=[
            pl.BlockSpec((TB, D), lambda i: (i, 0)),
            pl.BlockSpec((TB, D), lambda i: (i, 0)),
            pl.BlockSpec((TB, 1), lambda i: (i, 0)),
            pl.BlockSpec((TB, 1), lambda i: (i, 0)),
            pl.BlockSpec((1, D), lambda i: (0, 0)),
            pl.BlockSpec((D, D), lambda i: (0, 0)),
        ],
        out_specs=pl.BlockSpec((TB, D), lambda i: (i, 0)),
        out_shape=jax.ShapeDtypeStruct((S, D), jnp.float32),
    )(rowsA, rowsB, wA, wB, g_norm, Wo)


def kernel(hidden_states, Wq, Wk, Wv, Wg, Wo, g_norm):
    x = hidden_states[0]
    dstA, dstB, cmbA, cmbB, wA, wB, cvalid = _router(x, Wg)
    q = _qproj(x, Wq)
    gx, gq = _dispatch(x, q, dstA.reshape(S), dstB.reshape(S))
    o_exp = _attention(cvalid, gx, gq, Wk, Wv)
    rowsA, rowsB = _combine(o_exp, cmbA.reshape(S), cmbB.reshape(S))
    out = _final(rowsA, rowsB, wA.reshape(S, 1), wB.reshape(S, 1),
                 g_norm.reshape(1, D), Wo)
    return out[None]


# speedup vs baseline: 1.0490x; 1.0295x over previous
import functools

import jax
import jax.numpy as jnp
from jax import lax
from jax.experimental import pallas as pl
from jax.experimental.pallas import tpu as pltpu
from jax.experimental.pallas import tpu_sc as plsc

B, S, D = 1, 2048, 1024
E, TOPK, H = 8, 2, 8
DK = DV = D // H
CAP = S // TOPK
C = 256
N = CAP // C
EPS = 1e-5
PAD = 8
TRASH = E * CAP
NW = 32
TPW = S // NW
NEG = -1e30


def _elu1(x):
    return jnp.where(x > 0, x + 1.0, jnp.exp(jnp.minimum(x, 0.0)))


def _bf(x):
    return x.astype(jnp.bfloat16)


def _router_body(x_ref, wg_ref, wq_ref, dstA_ref, dstB_ref, cmbA_ref, cmbB_ref,
                 wA_ref, wB_ref, cval_ref, q_ref):
    q_ref[...] = _elu1(
        lax.dot_general(_bf(x_ref[...]), _bf(wq_ref[...]),
                        (((1,), (1,)), ((), ())),
                        preferred_element_type=jnp.float32))
    lg = lax.dot_general(wg_ref[...], x_ref[...],
                         (((1,), (1,)), ((), ())),
                         preferred_element_type=jnp.float32)
    ie = lax.broadcasted_iota(jnp.int32, (E, S), 0)
    m1 = jnp.max(lg, axis=0, keepdims=True)
    i1 = jnp.min(jnp.where(lg == m1, ie, E), axis=0, keepdims=True)
    lg2 = jnp.where(ie == i1, NEG, lg)
    m2 = jnp.max(lg2, axis=0, keepdims=True)
    i2 = jnp.min(jnp.where(lg2 == m2, ie, E), axis=0, keepdims=True)
    wA = 1.0 / (1.0 + jnp.exp(m2 - m1))
    wB = 1.0 - wA
    oh1 = (ie == i1)
    oh2 = (ie == i2)
    M = jnp.where(jnp.logical_or(oh1, oh2), 1, 0)
    inc = M
    sh = 1
    while sh < S:
        inc = inc + jnp.concatenate(
            [jnp.zeros((E, sh), jnp.int32), inc[:, : S - sh]], axis=1)
        sh *= 2
    exc = inc - M
    count = inc[:, S - 1 : S]
    over = jnp.maximum(count - CAP, 0)
    cval_ref[...] = jnp.minimum(count, CAP)
    oh1f = jnp.where(oh1, 1, 0)
    oh2f = jnp.where(oh2, 1, 0)
    r0 = jnp.sum(exc * oh1f, axis=0, keepdims=True)
    r1 = jnp.sum(exc * oh2f, axis=0, keepdims=True)
    ov0 = jnp.sum(over * oh1f, axis=0, keepdims=True)
    ov1 = jnp.sum(over * oh2f, axis=0, keepdims=True)
    s0 = r0 - ov0
    s1 = r1 - ov1
    d0 = i1 * CAP + s0
    d1 = i2 * CAP + s1
    val0 = s0 >= 0
    val1 = s1 >= 0
    dstA_ref[...] = jnp.where(val0, d0, TRASH)
    dstB_ref[...] = jnp.where(val1, d1, TRASH)
    cmbA_ref[...] = jnp.where(val0, d0, 0)
    cmbB_ref[...] = jnp.where(val1, d1, 0)
    wA_ref[...] = jnp.where(val0, wA, 0.0)
    wB_ref[...] = jnp.where(val1, wB, 0.0)


def _router(x, Wg, Wq):
    i32 = jnp.int32
    f32 = jnp.float32
    return pl.pallas_call(
        _router_body,
        out_shape=[
            jax.ShapeDtypeStruct((1, S), i32),
            jax.ShapeDtypeStruct((1, S), i32),
            jax.ShapeDtypeStruct((1, S), i32),
            jax.ShapeDtypeStruct((1, S), i32),
            jax.ShapeDtypeStruct((1, S), f32),
            jax.ShapeDtypeStruct((1, S), f32),
            jax.ShapeDtypeStruct((E, 1), i32),
            jax.ShapeDtypeStruct((S, D), f32),
        ],
    )(x, Wg, Wq)


def _dispatch_body(x_hbm, q_hbm, dstA_hbm, dstB_hbm, gx_hbm, gq_hbm,
                   idxA, idxB, rows, sem):
    wid = lax.axis_index("s") * 2 + lax.axis_index("c")
    base = wid * TPW
    pltpu.sync_copy(dstA_hbm.at[pl.ds(base, TPW)], idxA)
    pltpu.sync_copy(dstB_hbm.at[pl.ds(base, TPW)], idxB)
    pltpu.sync_copy(x_hbm.at[pl.ds(base, TPW)], rows)
    pltpu.async_copy(rows, gx_hbm.at[idxA], sem).wait()
    pltpu.async_copy(rows, gx_hbm.at[idxB], sem).wait()
    pltpu.sync_copy(q_hbm.at[pl.ds(base, TPW)], rows)
    pltpu.async_copy(rows, gq_hbm.at[idxA], sem).wait()
    pltpu.async_copy(rows, gq_hbm.at[idxB], sem).wait()


def _dispatch(x, q, dstA, dstB):
    mesh = plsc.VectorSubcoreMesh(core_axis_name="c", subcore_axis_name="s")
    i32 = jnp.int32
    f32 = jnp.float32
    fn = functools.partial(
        pl.kernel,
        out_type=[
            jax.ShapeDtypeStruct((E * CAP + PAD, D), f32),
            jax.ShapeDtypeStruct((E * CAP + PAD, D), f32),
        ],
        mesh=mesh,
        scratch_types=[
            pltpu.VMEM((TPW,), i32),
            pltpu.VMEM((TPW,), i32),
            pltpu.VMEM((TPW, D), f32),
            pltpu.SemaphoreType.DMA,
        ],
    )(_dispatch_body)
    return fn(x, q, dstA, dstB)


def _attn_body(cval_ref, gx_ref, gq_ref, wk_ref, wv_ref, o_ref, st_ref,
               wkb_ref, wvb_ref):
    n = pl.program_id(1)
    e = pl.program_id(0)
    v = cval_ref[e, 0]

    @pl.when(jnp.logical_or(n == 0, n * C < v))
    def _():
        @pl.when(n == 0)
        def _():
            st_ref[...] = jnp.zeros((H, DK, DV), jnp.float32)
            wkb_ref[...] = _bf(wk_ref[0])
            wvb_ref[...] = _bf(wv_ref[0])

        xb = _bf(gx_ref[...])
        k = _elu1(lax.dot_general(xb, wkb_ref[...], (((1,), (1,)), ((), ())),
                                  preferred_element_type=jnp.float32))
        vv = lax.dot_general(xb, wvb_ref[...], (((1,), (1,)), ((), ())),
                             preferred_element_type=jnp.float32)
        rv = (n * C + lax.broadcasted_iota(jnp.int32, (C, 1), 0)) < v
        k = _bf(jnp.where(rv, k, 0.0))
        vb = _bf(jnp.where(rv, vv, 0.0))
        qb = _bf(gq_ref[...])
        ri = lax.broadcasted_iota(jnp.int32, (C, C), 0)
        ci = lax.broadcasted_iota(jnp.int32, (C, C), 1)
        tri = ri >= ci
        outs = []
        for h in range(H):
            qh = qb[:, h * DK : (h + 1) * DK]
            kh = k[:, h * DK : (h + 1) * DK]
            vh = vb[:, h * DV : (h + 1) * DV]
            sth = st_ref[h]
            inter = lax.dot_general(qh, _bf(sth), (((1,), (0,)), ((), ())),
                                    preferred_element_type=jnp.float32)
            attn = lax.dot_general(qh, kh, (((1,), (1,)), ((), ())),
                                   preferred_element_type=jnp.float32)
            attn = _bf(jnp.where(tri, attn, 0.0))
            intra = lax.dot_general(attn, vh, (((1,), (0,)), ((), ())),
                                    preferred_element_type=jnp.float32)
            outs.append(inter + intra)
            st_ref[h] = sth + lax.dot_general(
                kh, vh, (((0,), (0,)), ((), ())),
                preferred_element_type=jnp.float32)
        o_ref[...] = jnp.concatenate(outs, axis=1)


def _attention(cvalid, gx, gq, Wk, Wv):
    return pl.pallas_call(
        _attn_body,
        grid=(E, N),
        in_specs=[
            pl.BlockSpec(memory_space=pltpu.SMEM),
            pl.BlockSpec((C, D), lambda e, n: (e * N + n, 0)),
            pl.BlockSpec((C, D), lambda e, n: (e * N + n, 0)),
            pl.BlockSpec((1, D, D), lambda e, n: (e, 0, 0)),
            pl.BlockSpec((1, D, D), lambda e, n: (e, 0, 0)),
        ],
        out_specs=pl.BlockSpec((C, D), lambda e, n: (e * N + n, 0)),
        out_shape=jax.ShapeDtypeStruct((E * CAP, D), jnp.float32),
        scratch_shapes=[pltpu.VMEM((H, DK, DV), jnp.float32),
                        pltpu.VMEM((D, D), jnp.bfloat16),
                        pltpu.VMEM((D, D), jnp.bfloat16)],
    )(cvalid, gx, gq, Wk, Wv)


def _combine_body(o_hbm, cmbA_hbm, cmbB_hbm, rA_hbm, rB_hbm, idx, rows, sem):
    wid = lax.axis_index("s") * 2 + lax.axis_index("c")
    base = wid * TPW
    pltpu.sync_copy(cmbA_hbm.at[pl.ds(base, TPW)], idx)
    pltpu.async_copy(o_hbm.at[idx], rows, sem).wait()
    pltpu.sync_copy(rows, rA_hbm.at[pl.ds(base, TPW)])
    pltpu.sync_copy(cmbB_hbm.at[pl.ds(base, TPW)], idx)
    pltpu.async_copy(o_hbm.at[idx], rows, sem).wait()
    pltpu.sync_copy(rows, rB_hbm.at[pl.ds(base, TPW)])


def _combine(o_exp, cmbA, cmbB):
    mesh = plsc.VectorSubcoreMesh(core_axis_name="c", subcore_axis_name="s")
    i32 = jnp.int32
    f32 = jnp.float32
    fn = functools.partial(
        pl.kernel,
        out_type=[
            jax.ShapeDtypeStruct((S, D), f32),
            jax.ShapeDtypeStruct((S, D), f32),
        ],
        mesh=mesh,
        scratch_types=[
            pltpu.VMEM((TPW,), i32),
            pltpu.VMEM((TPW, D), f32),
            pltpu.SemaphoreType.DMA,
        ],
    )(_combine_body)
    return fn(o_exp, cmbA, cmbB)


def _final_body(ra_ref, rb_ref, wa_ref, wb_ref, g_ref, wo_ref, out_ref):
    y = ra_ref[...] * wa_ref[...] + rb_ref[...] * wb_ref[...]
    ms = jnp.mean(y * y, axis=1, keepdims=True)
    y = y * lax.rsqrt(ms + EPS) * g_ref[...]
    out_ref[...] = lax.dot_general(_bf(y), _bf(wo_ref[...]),
                                   (((1,), (1,)), ((), ())),
                                   preferred_element_type=jnp.float32)


def _final(rowsA, rowsB, wA, wB, g_norm, Wo):
    TB = 256
    return pl.pallas_call(
        _final_body,
        grid=(S // TB,),
        in_specs=[
            pl.BlockSpec((TB, D), lambda i: (i, 0)),
            pl.BlockSpec((TB, D), lambda i: (i, 0)),
            pl.BlockSpec((TB, 1), lambda i: (i, 0)),
            pl.BlockSpec((TB, 1), lambda i: (i, 0)),
            pl.BlockSpec((1, D), lambda i: (0, 0)),
            pl.BlockSpec((D, D), lambda i: (0, 0)),
        ],
        out_specs=pl.BlockSpec((TB, D), lambda i: (i, 0)),
        out_shape=jax.ShapeDtypeStruct((S, D), jnp.float32),
    )(rowsA, rowsB, wA, wB, g_norm, Wo)


def kernel(hidden_states, Wq, Wk, Wv, Wg, Wo, g_norm):
    x = hidden_states[0]
    dstA, dstB, cmbA, cmbB, wA, wB, cvalid, q = _router(x, Wg, Wq)
    gx, gq = _dispatch(x, q, dstA.reshape(S), dstB.reshape(S))
    o_exp = _attention(cvalid, gx, gq, Wk, Wv)
    rowsA, rowsB = _combine(o_exp, cmbA.reshape(S), cmbB.reshape(S))
    out = _final(rowsA, rowsB, wA.reshape(S, 1), wB.reshape(S, 1),
                 g_norm.reshape(1, D), Wo)
    return out[None]


# speedup vs baseline: 1.0553x; 1.0060x over previous
import functools

import jax
import jax.numpy as jnp
from jax import lax
from jax.experimental import pallas as pl
from jax.experimental.pallas import tpu as pltpu
from jax.experimental.pallas import tpu_sc as plsc

B, S, D = 1, 2048, 1024
E, TOPK, H = 8, 2, 8
DK = DV = D // H
CAP = S // TOPK
C = 256
N = CAP // C
EPS = 1e-5
PAD = 8
TRASH = E * CAP
NW = 32
TPW = S // NW
NEG = -1e30


def _elu1(x):
    return jnp.where(x > 0, x + 1.0, jnp.exp(jnp.minimum(x, 0.0)))


def _bf(x):
    return x.astype(jnp.bfloat16)


def _router_body(x_ref, wg_ref, wq_ref, dstA_ref, dstB_ref, cmbA_ref, cmbB_ref,
                 wA_ref, wB_ref, cval_ref, q_ref):
    q_ref[...] = _elu1(
        lax.dot_general(_bf(x_ref[...]), _bf(wq_ref[...]),
                        (((1,), (1,)), ((), ())),
                        preferred_element_type=jnp.float32))
    lg = lax.dot_general(wg_ref[...], x_ref[...],
                         (((1,), (1,)), ((), ())),
                         preferred_element_type=jnp.float32)
    ie = lax.broadcasted_iota(jnp.int32, (E, S), 0)
    m1 = jnp.max(lg, axis=0, keepdims=True)
    i1 = jnp.min(jnp.where(lg == m1, ie, E), axis=0, keepdims=True)
    lg2 = jnp.where(ie == i1, NEG, lg)
    m2 = jnp.max(lg2, axis=0, keepdims=True)
    i2 = jnp.min(jnp.where(lg2 == m2, ie, E), axis=0, keepdims=True)
    wA = 1.0 / (1.0 + jnp.exp(m2 - m1))
    wB = 1.0 - wA
    oh1 = (ie == i1)
    oh2 = (ie == i2)
    M = jnp.where(jnp.logical_or(oh1, oh2), 1, 0)
    inc = M
    sh = 1
    while sh < S:
        inc = inc + jnp.concatenate(
            [jnp.zeros((E, sh), jnp.int32), inc[:, : S - sh]], axis=1)
        sh *= 2
    exc = inc - M
    count = inc[:, S - 1 : S]
    over = jnp.maximum(count - CAP, 0)
    cval_ref[...] = jnp.minimum(count, CAP)
    oh1f = jnp.where(oh1, 1, 0)
    oh2f = jnp.where(oh2, 1, 0)
    r0 = jnp.sum(exc * oh1f, axis=0, keepdims=True)
    r1 = jnp.sum(exc * oh2f, axis=0, keepdims=True)
    ov0 = jnp.sum(over * oh1f, axis=0, keepdims=True)
    ov1 = jnp.sum(over * oh2f, axis=0, keepdims=True)
    s0 = r0 - ov0
    s1 = r1 - ov1
    d0 = i1 * CAP + s0
    d1 = i2 * CAP + s1
    val0 = s0 >= 0
    val1 = s1 >= 0
    dstA_ref[...] = jnp.where(val0, d0, TRASH)
    dstB_ref[...] = jnp.where(val1, d1, TRASH)
    cmbA_ref[...] = jnp.where(val0, d0, 0)
    cmbB_ref[...] = jnp.where(val1, d1, 0)
    wA_ref[...] = jnp.where(val0, wA, 0.0)
    wB_ref[...] = jnp.where(val1, wB, 0.0)


def _router(x, Wg, Wq):
    i32 = jnp.int32
    f32 = jnp.float32
    return pl.pallas_call(
        _router_body,
        out_shape=[
            jax.ShapeDtypeStruct((1, S), i32),
            jax.ShapeDtypeStruct((1, S), i32),
            jax.ShapeDtypeStruct((1, S), i32),
            jax.ShapeDtypeStruct((1, S), i32),
            jax.ShapeDtypeStruct((1, S), f32),
            jax.ShapeDtypeStruct((1, S), f32),
            jax.ShapeDtypeStruct((E, 1), i32),
            jax.ShapeDtypeStruct((S, D), f32),
        ],
    )(x, Wg, Wq)


def _dispatch_body(x_hbm, q_hbm, dstA_hbm, dstB_hbm, gx_hbm, gq_hbm,
                   idxA, idxB, rows, sem):
    wid = lax.axis_index("s") * 2 + lax.axis_index("c")
    base = wid * TPW
    pltpu.sync_copy(dstA_hbm.at[pl.ds(base, TPW)], idxA)
    pltpu.sync_copy(dstB_hbm.at[pl.ds(base, TPW)], idxB)
    pltpu.sync_copy(x_hbm.at[pl.ds(base, TPW)], rows)
    cx1 = pltpu.async_copy(rows, gx_hbm.at[idxA], sem)
    cx2 = pltpu.async_copy(rows, gx_hbm.at[idxB], sem)
    cx1.wait()
    cx2.wait()
    pltpu.sync_copy(q_hbm.at[pl.ds(base, TPW)], rows)
    cq1 = pltpu.async_copy(rows, gq_hbm.at[idxA], sem)
    cq2 = pltpu.async_copy(rows, gq_hbm.at[idxB], sem)
    cq1.wait()
    cq2.wait()


def _dispatch(x, q, dstA, dstB):
    mesh = plsc.VectorSubcoreMesh(core_axis_name="c", subcore_axis_name="s")
    i32 = jnp.int32
    f32 = jnp.float32
    fn = functools.partial(
        pl.kernel,
        out_type=[
            jax.ShapeDtypeStruct((E * CAP + PAD, D), f32),
            jax.ShapeDtypeStruct((E * CAP + PAD, D), f32),
        ],
        mesh=mesh,
        scratch_types=[
            pltpu.VMEM((TPW,), i32),
            pltpu.VMEM((TPW,), i32),
            pltpu.VMEM((TPW, D), f32),
            pltpu.SemaphoreType.DMA,
        ],
    )(_dispatch_body)
    return fn(x, q, dstA, dstB)


def _attn_body(cval_ref, gx_ref, gq_ref, wk_ref, wv_ref, o_ref, st_ref,
               wkb_ref, wvb_ref):
    n = pl.program_id(1)
    e = pl.program_id(0)
    v = cval_ref[e, 0]

    @pl.when(jnp.logical_or(n == 0, n * C < v))
    def _():
        @pl.when(n == 0)
        def _():
            st_ref[...] = jnp.zeros((H, DK, DV), jnp.float32)
            wkb_ref[...] = _bf(wk_ref[0])
            wvb_ref[...] = _bf(wv_ref[0])

        xb = _bf(gx_ref[...])
        k = _elu1(lax.dot_general(xb, wkb_ref[...], (((1,), (1,)), ((), ())),
                                  preferred_element_type=jnp.float32))
        vv = lax.dot_general(xb, wvb_ref[...], (((1,), (1,)), ((), ())),
                             preferred_element_type=jnp.float32)
        rv = (n * C + lax.broadcasted_iota(jnp.int32, (C, 1), 0)) < v
        k = _bf(jnp.where(rv, k, 0.0))
        vb = _bf(jnp.where(rv, vv, 0.0))
        qb = _bf(gq_ref[...])
        ri = lax.broadcasted_iota(jnp.int32, (C, C), 0)
        ci = lax.broadcasted_iota(jnp.int32, (C, C), 1)
        tri = ri >= ci
        outs = []
        for h in range(H):
            qh = qb[:, h * DK : (h + 1) * DK]
            kh = k[:, h * DK : (h + 1) * DK]
            vh = vb[:, h * DV : (h + 1) * DV]
            sth = st_ref[h]
            inter = lax.dot_general(qh, _bf(sth), (((1,), (0,)), ((), ())),
                                    preferred_element_type=jnp.float32)
            attn = lax.dot_general(qh, kh, (((1,), (1,)), ((), ())),
                                   preferred_element_type=jnp.float32)
            attn = _bf(jnp.where(tri, attn, 0.0))
            intra = lax.dot_general(attn, vh, (((1,), (0,)), ((), ())),
                                    preferred_element_type=jnp.float32)
            outs.append(inter + intra)
            st_ref[h] = sth + lax.dot_general(
                kh, vh, (((0,), (0,)), ((), ())),
                preferred_element_type=jnp.float32)
        o_ref[...] = jnp.concatenate(outs, axis=1)


def _attention(cvalid, gx, gq, Wk, Wv):
    return pl.pallas_call(
        _attn_body,
        grid=(E, N),
        in_specs=[
            pl.BlockSpec(memory_space=pltpu.SMEM),
            pl.BlockSpec((C, D), lambda e, n: (e * N + n, 0)),
            pl.BlockSpec((C, D), lambda e, n: (e * N + n, 0)),
            pl.BlockSpec((1, D, D), lambda e, n: (e, 0, 0)),
            pl.BlockSpec((1, D, D), lambda e, n: (e, 0, 0)),
        ],
        out_specs=pl.BlockSpec((C, D), lambda e, n: (e * N + n, 0)),
        out_shape=jax.ShapeDtypeStruct((E * CAP, D), jnp.float32),
        scratch_shapes=[pltpu.VMEM((H, DK, DV), jnp.float32),
                        pltpu.VMEM((D, D), jnp.bfloat16),
                        pltpu.VMEM((D, D), jnp.bfloat16)],
    )(cvalid, gx, gq, Wk, Wv)


def _combine_body(o_hbm, cmbA_hbm, cmbB_hbm, rA_hbm, rB_hbm, idx, rows, sem):
    wid = lax.axis_index("s") * 2 + lax.axis_index("c")
    base = wid * TPW
    pltpu.sync_copy(cmbA_hbm.at[pl.ds(base, TPW)], idx)
    pltpu.async_copy(o_hbm.at[idx], rows, sem).wait()
    pltpu.sync_copy(rows, rA_hbm.at[pl.ds(base, TPW)])
    pltpu.sync_copy(cmbB_hbm.at[pl.ds(base, TPW)], idx)
    pltpu.async_copy(o_hbm.at[idx], rows, sem).wait()
    pltpu.sync_copy(rows, rB_hbm.at[pl.ds(base, TPW)])


def _combine(o_exp, cmbA, cmbB):
    mesh = plsc.VectorSubcoreMesh(core_axis_name="c", subcore_axis_name="s")
    i32 = jnp.int32
    f32 = jnp.float32
    fn = functools.partial(
        pl.kernel,
        out_type=[
            jax.ShapeDtypeStruct((S, D), f32),
            jax.ShapeDtypeStruct((S, D), f32),
        ],
        mesh=mesh,
        scratch_types=[
            pltpu.VMEM((TPW,), i32),
            pltpu.VMEM((TPW, D), f32),
            pltpu.SemaphoreType.DMA,
        ],
    )(_combine_body)
    return fn(o_exp, cmbA, cmbB)


def _final_body(ra_ref, rb_ref, wa_ref, wb_ref, g_ref, wo_ref, out_ref):
    y = ra_ref[...] * wa_ref[...] + rb_ref[...] * wb_ref[...]
    ms = jnp.mean(y * y, axis=1, keepdims=True)
    y = y * lax.rsqrt(ms + EPS) * g_ref[...]
    out_ref[...] = lax.dot_general(_bf(y), _bf(wo_ref[...]),
                                   (((1,), (1,)), ((), ())),
                                   preferred_element_type=jnp.float32)


def _final(rowsA, rowsB, wA, wB, g_norm, Wo):
    TB = 256
    return pl.pallas_call(
        _final_body,
        grid=(S // TB,),
        in_specs=[
            pl.BlockSpec((TB, D), lambda i: (i, 0)),
            pl.BlockSpec((TB, D), lambda i: (i, 0)),
            pl.BlockSpec((TB, 1), lambda i: (i, 0)),
            pl.BlockSpec((TB, 1), lambda i: (i, 0)),
            pl.BlockSpec((1, D), lambda i: (0, 0)),
            pl.BlockSpec((D, D), lambda i: (0, 0)),
        ],
        out_specs=pl.BlockSpec((TB, D), lambda i: (i, 0)),
        out_shape=jax.ShapeDtypeStruct((S, D), jnp.float32),
    )(rowsA, rowsB, wA, wB, g_norm, Wo)


def kernel(hidden_states, Wq, Wk, Wv, Wg, Wo, g_norm):
    x = hidden_states[0]
    dstA, dstB, cmbA, cmbB, wA, wB, cvalid, q = _router(x, Wg, Wq)
    gx, gq = _dispatch(x, q, dstA.reshape(S), dstB.reshape(S))
    o_exp = _attention(cvalid, gx, gq, Wk, Wv)
    rowsA, rowsB = _combine(o_exp, cmbA.reshape(S), cmbB.reshape(S))
    out = _final(rowsA, rowsB, wA.reshape(S, 1), wB.reshape(S, 1),
                 g_norm.reshape(1, D), Wo)
    return out[None]
